# Initial kernel scaffold; baseline (speedup 1.0000x reference)
import jax
import jax.numpy as jnp
from jax.experimental import pallas as pl


def kernel(x, edge_index, W_in, b_in, blk0_W1, blk0_b1, blk0_W2, blk0_b2, blk0_W3, blk0_b3, blk1_W1, blk1_b1, blk1_W2, blk1_b2, blk1_W3, blk1_b3, W_out, b_out):
    raise NotImplementedError("write your pallas kernel here")


# speedup vs baseline: 1.8053x; 1.8053x over previous
import functools

import jax
import jax.numpy as jnp
from jax import lax
from jax.experimental import pallas as pl
from jax.experimental.pallas import tpu as pltpu
from jax.experimental.pallas import tpu_sc as plsc

_N = 10000
_E = 160000
_H = 512
_NC = 2
_NS = 16
_NW = _NC * _NS
_GCH = 64
_GK = 80
_SCH = 128
_SK = 40
_EW = _SK * _SCH
_EP = _NW * _EW
_HC = 128
_NPASS = _H // _HC
_NROW = 632
_NRL = _N - 15 * _NROW
_NACC = _N + 16
_TE = 2048
_TM = 2000


def _sc_mesh():
    return plsc.VectorSubcoreMesh(core_axis_name="c", subcore_axis_name="s")


def _mm1_body(p0, w, b, o):
    o[...] = jnp.dot(p0[...], w[...], preferred_element_type=jnp.float32) + b[...]


def _mm2_body(p0, p1, w, b, o):
    h = p0[...] + p1[...]
    o[...] = jnp.dot(h, w[...], preferred_element_type=jnp.float32) + b[...]


def _mm_bias(parts, W, bvec):
    M, K = parts[0].shape
    Nc = W.shape[1]
    body = _mm1_body if len(parts) == 1 else _mm2_body
    part_specs = [pl.BlockSpec((_TM, K), lambda i: (i, 0)) for _ in parts]
    return pl.pallas_call(
        body,
        grid=(M // _TM,),
        in_specs=part_specs + [
            pl.BlockSpec((K, Nc), lambda i: (0, 0)),
            pl.BlockSpec((1, Nc), lambda i: (0, 0)),
        ],
        out_specs=pl.BlockSpec((_TM, Nc), lambda i: (i, 0)),
        out_shape=jax.ShapeDtypeStruct((M, Nc), jnp.float32),
    )(*parts, W, bvec.reshape(1, Nc))


def _ab1_body(p0, wt, wb, b1, a, b):
    h = p0[...]
    a[...] = jnp.dot(h, wt[...], preferred_element_type=jnp.float32) + b1[...]
    b[...] = jnp.dot(h, wb[...], preferred_element_type=jnp.float32)


def _ab2_body(p0, p1, wt, wb, b1, a, b):
    h = p0[...] + p1[...]
    a[...] = jnp.dot(h, wt[...], preferred_element_type=jnp.float32) + b1[...]
    b[...] = jnp.dot(h, wb[...], preferred_element_type=jnp.float32)


def _mm_ab(parts, W1, b1):
    M, K = parts[0].shape
    body = _ab1_body if len(parts) == 1 else _ab2_body
    part_specs = [pl.BlockSpec((_TM, K), lambda i: (i, 0)) for _ in parts]
    out_sds = jax.ShapeDtypeStruct((M, _H), jnp.float32)
    return pl.pallas_call(
        body,
        grid=(M // _TM,),
        in_specs=part_specs + [
            pl.BlockSpec((K, _H), lambda i: (0, 0)),
            pl.BlockSpec((K, _H), lambda i: (0, 0)),
            pl.BlockSpec((1, _H), lambda i: (0, 0)),
        ],
        out_specs=[pl.BlockSpec((_TM, _H), lambda i: (i, 0)),
                   pl.BlockSpec((_TM, _H), lambda i: (i, 0))],
        out_shape=[out_sds, out_sds],
    )(*parts, W1[:_H], W1[_H:], b1.reshape(1, _H))


def _mlp_body(aj, bi, w2, b2, w3, b3, v):
    t = jnp.maximum(aj[...] + bi[...], 0.0)
    u = jnp.dot(t, w2[...], preferred_element_type=jnp.float32) + b2[...]
    u = jnp.maximum(u, 0.0)
    v[...] = jnp.dot(u, w3[...], preferred_element_type=jnp.float32) + b3[...]


def _edge_mlp(aj, bi, W2, b2, W3, b3):
    return pl.pallas_call(
        _mlp_body,
        grid=(_EP // _TE,),
        in_specs=[
            pl.BlockSpec((_TE, _H), lambda i: (i, 0)),
            pl.BlockSpec((_TE, _H), lambda i: (i, 0)),
            pl.BlockSpec((_H, _H), lambda i: (0, 0)),
            pl.BlockSpec((1, _H), lambda i: (0, 0)),
            pl.BlockSpec((_H, _H), lambda i: (0, 0)),
            pl.BlockSpec((1, _H), lambda i: (0, 0)),
        ],
        out_specs=pl.BlockSpec((_TE, _H), lambda i: (i, 0)),
        out_shape=jax.ShapeDtypeStruct((_EP, _H), jnp.float32),
    )(aj, bi, W2, b2.reshape(1, _H), W3, b3.reshape(1, _H))


def _gather_body(a_hbm, b_hbm, srcg, dstg, aj_hbm, bi_hbm,
                 idx_v, r0, r1, s0, s1):
    c = lax.axis_index("c")
    s = lax.axis_index("s")
    w = s * _NC + c
    base = w * _EW
    bufs = (r0, r1)
    sems = (s0, s1)

    def run(table, idx_src, out):
        pltpu.sync_copy(idx_src.at[w], idx_v)
        pltpu.async_copy(table.at[idx_v.at[0]], bufs[0], sems[0])

        def outer(g):
            for par in range(2):
                j = g + par
                nb = 1 - par

                @pl.when(j + 1 < _GK)
                def _start_next():
                    pltpu.async_copy(table.at[idx_v.at[j + 1]], bufs[nb], sems[nb])

                pltpu.make_async_copy(table.at[idx_v.at[j]], bufs[par], sems[par]).wait()
                pltpu.sync_copy(bufs[par], out.at[pl.ds(base + j * _GCH, _GCH)])

        pl.loop(0, _GK, step=2)(outer)

    run(a_hbm, srcg, aj_hbm)
    run(b_hbm, dstg, bi_hbm)


def _sc_gather(A, B, srcg, dstg):
    out_sds = jax.ShapeDtypeStruct((_EP, _H), jnp.float32)
    k = functools.partial(
        pl.kernel,
        out_type=(out_sds, out_sds),
        mesh=_sc_mesh(),
        scratch_types=[
            pltpu.VMEM((_GK, _GCH), jnp.int32),
            pltpu.VMEM((_GCH, _H), jnp.float32),
            pltpu.VMEM((_GCH, _H), jnp.float32),
            pltpu.SemaphoreType.DMA,
            pltpu.SemaphoreType.DMA,
        ],
    )(_gather_body)
    return k(A, B, srcg, dstg)


def _scatter_body(v_hbm, dsts, zeros_hbm, out_hbm,
                  idx_v, vb0, vb1, acc, s0, s1):
    c = lax.axis_index("c")
    s = lax.axis_index("s")
    w = s * _NC + c
    base = w * _EW
    pltpu.sync_copy(dsts.at[w], idx_v)
    vbufs = (vb0, vb1)
    sems = (s0, s1)

    for p in range(_NPASS):
        @pl.when(s < _NS - 1)
        def _zero_main():
            pltpu.sync_copy(zeros_hbm, acc.at[pl.ds(s * _NROW, _NROW)])

        @pl.when(s == _NS - 1)
        def _zero_last():
            pltpu.sync_copy(zeros_hbm.at[pl.ds(0, _NRL + 16)],
                            acc.at[pl.ds((_NS - 1) * _NROW, _NRL + 16)])

        plsc.subcore_barrier()

        pltpu.async_copy(v_hbm.at[pl.ds(base, _SCH), pl.ds(p * _HC, _HC)],
                         vbufs[0], sems[0])

        def outer(g):
            for par in range(2):
                j = g + par
                nb = 1 - par

                @pl.when(j + 1 < _SK)
                def _start_next():
                    pltpu.async_copy(
                        v_hbm.at[pl.ds(base + (j + 1) * _SCH, _SCH),
                                 pl.ds(p * _HC, _HC)],
                        vbufs[nb], sems[nb])

                pltpu.make_async_copy(
                    v_hbm.at[pl.ds(base + j * _SCH, _SCH), pl.ds(p * _HC, _HC)],
                    vbufs[par], sems[par]).wait()
                pltpu.sync_copy(vbufs[par], acc.at[idx_v.at[j]], add=True)

        pl.loop(0, _SK, step=2)(outer)
        plsc.subcore_barrier()

        @pl.when(s < _NS - 1)
        def _drain_main():
            pltpu.sync_copy(acc.at[pl.ds(s * _NROW, _NROW)],
                            out_hbm.at[c, pl.ds(s * _NROW, _NROW),
                                       pl.ds(p * _HC, _HC)])

        @pl.when(s == _NS - 1)
        def _drain_last():
            pltpu.sync_copy(acc.at[pl.ds((_NS - 1) * _NROW, _NRL)],
                            out_hbm.at[c, pl.ds((_NS - 1) * _NROW, _NRL),
                                       pl.ds(p * _HC, _HC)])

        plsc.subcore_barrier()


def _sc_scatter(v, dsts, zeros_hbm):
    k = functools.partial(
        pl.kernel,
        out_type=jax.ShapeDtypeStruct((_NC, _N, _H), jnp.float32),
        mesh=_sc_mesh(),
        scratch_types=[
            pltpu.VMEM((_SK, _SCH), jnp.int32),
            pltpu.VMEM((_SCH, _HC), jnp.float32),
            pltpu.VMEM((_SCH, _HC), jnp.float32),
            pltpu.VMEM_SHARED((_NACC, _HC), jnp.float32),
            pltpu.SemaphoreType.DMA,
            pltpu.SemaphoreType.DMA,
        ],
    )(_scatter_body)
    return k(v, dsts, zeros_hbm)


def kernel(x, edge_index, W_in, b_in,
           blk0_W1, blk0_b1, blk0_W2, blk0_b2, blk0_W3, blk0_b3,
           blk1_W1, blk1_b1, blk1_W2, blk1_b2, blk1_W3, blk1_b3,
           W_out, b_out):
    src = edge_index[0]
    dst = edge_index[1]
    pad = _EP - _E
    srcg = jnp.concatenate([src, jnp.zeros((pad,), jnp.int32)]).reshape(_NW, _GK, _GCH)
    dstg = jnp.concatenate([dst, jnp.zeros((pad,), jnp.int32)]).reshape(_NW, _GK, _GCH)
    dsts = jnp.concatenate([dst, jnp.full((pad,), _N, jnp.int32)]).reshape(_NW, _SK, _SCH)
    zeros_hbm = jnp.zeros((_NROW, _HC), jnp.float32)

    h0 = _mm_bias([x], W_in, b_in)

    parts = None
    for (W1, b1, W2, b2, W3, b3) in (
            (blk0_W1, blk0_b1, blk0_W2, blk0_b2, blk0_W3, blk0_b3),
            (blk1_W1, blk1_b1, blk1_W2, blk1_b2, blk1_W3, blk1_b3)):
        hin = [h0] if parts is None else [parts[0], parts[1]]
        A, B = _mm_ab(hin, W1, b1)
        aj, bi = _sc_gather(A, B, srcg, dstg)
        v = _edge_mlp(aj, bi, W2, b2, W3, b3)
        parts = _sc_scatter(v, dsts, zeros_hbm)

    return _mm_bias([parts[0], parts[1]], W_out, b_out)


# speedup vs baseline: 2.2187x; 1.2290x over previous
import functools

import jax
import jax.numpy as jnp
from jax import lax
from jax.experimental import pallas as pl
from jax.experimental.pallas import tpu as pltpu
from jax.experimental.pallas import tpu_sc as plsc

_N = 10000
_E = 160000
_H = 512
_NC = 2
_NS = 16
_NW = _NC * _NS
_GCH = 64
_GK = 80
_SCH = 128
_SK = 40
_EW = _SK * _SCH
_EP = _NW * _EW
_HC = 128
_NPASS = _H // _HC
_NROW = 632
_NRL = _N - 15 * _NROW
_NACC = _N + 16
_TE = 2048
_TM = 2000


def _sc_mesh():
    return plsc.VectorSubcoreMesh(core_axis_name="c", subcore_axis_name="s")


def _mm1_body(p0, w, b, o):
    o[...] = jnp.dot(p0[...], w[...], preferred_element_type=jnp.float32) + b[...]


def _mm2_body(p0, p1, w, b, o):
    h = p0[...] + p1[...]
    o[...] = jnp.dot(h, w[...], preferred_element_type=jnp.float32) + b[...]


def _mm_bias(parts, W, bvec):
    M, K = parts[0].shape
    Nc = W.shape[1]
    body = _mm1_body if len(parts) == 1 else _mm2_body
    part_specs = [pl.BlockSpec((_TM, K), lambda i: (i, 0)) for _ in parts]
    return pl.pallas_call(
        body,
        grid=(M // _TM,),
        in_specs=part_specs + [
            pl.BlockSpec((K, Nc), lambda i: (0, 0)),
            pl.BlockSpec((1, Nc), lambda i: (0, 0)),
        ],
        out_specs=pl.BlockSpec((_TM, Nc), lambda i: (i, 0)),
        out_shape=jax.ShapeDtypeStruct((M, Nc), jnp.float32),
    )(*parts, W, bvec.reshape(1, Nc))


def _pack_bf16(x32):
    half = x32.shape[1] // 2
    u = jax.lax.bitcast_convert_type(x32.astype(jnp.bfloat16),
                                     jnp.uint16).astype(jnp.uint32)
    w = u[:, :half] | (u[:, half:] << 16)
    return jax.lax.bitcast_convert_type(w, jnp.int32)


def _unpack_bf16(p):
    u = jax.lax.bitcast_convert_type(p, jnp.uint32)
    lo = jax.lax.bitcast_convert_type((u & 0xFFFF).astype(jnp.uint16),
                                      jnp.bfloat16)
    hi = jax.lax.bitcast_convert_type((u >> 16).astype(jnp.uint16),
                                      jnp.bfloat16)
    return lo, hi


def _ab1_body(p0, wt, wb, b1, a, b):
    h = p0[...]
    a[...] = _pack_bf16(jnp.dot(h, wt[...],
                                preferred_element_type=jnp.float32) + b1[...])
    b[...] = _pack_bf16(jnp.dot(h, wb[...],
                                preferred_element_type=jnp.float32))


def _ab2_body(p0, p1, wt, wb, b1, a, b):
    h = p0[...] + p1[...]
    a[...] = _pack_bf16(jnp.dot(h, wt[...],
                                preferred_element_type=jnp.float32) + b1[...])
    b[...] = _pack_bf16(jnp.dot(h, wb[...],
                                preferred_element_type=jnp.float32))


def _mm_ab(parts, W1, b1):
    M, K = parts[0].shape
    body = _ab1_body if len(parts) == 1 else _ab2_body
    part_specs = [pl.BlockSpec((_TM, K), lambda i: (i, 0)) for _ in parts]
    out_sds = jax.ShapeDtypeStruct((M, _H // 2), jnp.int32)
    return pl.pallas_call(
        body,
        grid=(M // _TM,),
        in_specs=part_specs + [
            pl.BlockSpec((K, _H), lambda i: (0, 0)),
            pl.BlockSpec((K, _H), lambda i: (0, 0)),
            pl.BlockSpec((1, _H), lambda i: (0, 0)),
        ],
        out_specs=[pl.BlockSpec((_TM, _H // 2), lambda i: (i, 0)),
                   pl.BlockSpec((_TM, _H // 2), lambda i: (i, 0))],
        out_shape=[out_sds, out_sds],
    )(*parts, W1[:_H], W1[_H:], b1.reshape(1, _H))


def _mlp_body(aj, bi, w2, b2, w3, b3, v):
    aj_lo, aj_hi = _unpack_bf16(aj[...])
    bi_lo, bi_hi = _unpack_bf16(bi[...])
    t_lo = jnp.maximum(aj_lo + bi_lo, jnp.bfloat16(0.0))
    t_hi = jnp.maximum(aj_hi + bi_hi, jnp.bfloat16(0.0))
    w2v = w2[...]
    u = (jnp.dot(t_lo, w2v[:_H // 2], preferred_element_type=jnp.float32)
         + jnp.dot(t_hi, w2v[_H // 2:], preferred_element_type=jnp.float32)
         + b2[...])
    u = jnp.maximum(u, 0.0).astype(jnp.bfloat16)
    v[...] = jnp.dot(u, w3[...], preferred_element_type=jnp.float32) + b3[...]


def _edge_mlp(aj, bi, W2, b2, W3, b3):
    return pl.pallas_call(
        _mlp_body,
        grid=(_EP // _TE,),
        in_specs=[
            pl.BlockSpec((_TE, _H // 2), lambda i: (i, 0)),
            pl.BlockSpec((_TE, _H // 2), lambda i: (i, 0)),
            pl.BlockSpec((_H, _H), lambda i: (0, 0)),
            pl.BlockSpec((1, _H), lambda i: (0, 0)),
            pl.BlockSpec((_H, _H), lambda i: (0, 0)),
            pl.BlockSpec((1, _H), lambda i: (0, 0)),
        ],
        out_specs=pl.BlockSpec((_TE, _H), lambda i: (i, 0)),
        out_shape=jax.ShapeDtypeStruct((_EP, _H), jnp.float32),
    )(aj, bi, W2.astype(jnp.bfloat16), b2.reshape(1, _H),
      W3.astype(jnp.bfloat16), b3.reshape(1, _H))


def _gather_body(a_hbm, b_hbm, srcg, dstg, aj_hbm, bi_hbm,
                 idx_v, r0, r1, s0, s1):
    c = lax.axis_index("c")
    s = lax.axis_index("s")
    w = s * _NC + c
    base = w * _EW
    bufs = (r0, r1)
    sems = (s0, s1)

    def run(table, idx_src, out):
        pltpu.sync_copy(idx_src.at[w], idx_v)
        pltpu.async_copy(table.at[idx_v.at[0]], bufs[0], sems[0])

        def outer(g):
            for par in range(2):
                j = g + par
                nb = 1 - par

                @pl.when(j + 1 < _GK)
                def _start_next():
                    pltpu.async_copy(table.at[idx_v.at[j + 1]], bufs[nb], sems[nb])

                pltpu.make_async_copy(table.at[idx_v.at[j]], bufs[par], sems[par]).wait()
                pltpu.sync_copy(bufs[par], out.at[pl.ds(base + j * _GCH, _GCH)])

        pl.loop(0, _GK, step=2)(outer)

    run(a_hbm, srcg, aj_hbm)
    run(b_hbm, dstg, bi_hbm)


def _sc_gather(A, B, srcg, dstg):
    out_sds = jax.ShapeDtypeStruct((_EP, _H // 2), jnp.int32)
    k = functools.partial(
        pl.kernel,
        out_type=(out_sds, out_sds),
        mesh=_sc_mesh(),
        scratch_types=[
            pltpu.VMEM((_GK, _GCH), jnp.int32),
            pltpu.VMEM((_GCH, _H // 2), jnp.int32),
            pltpu.VMEM((_GCH, _H // 2), jnp.int32),
            pltpu.SemaphoreType.DMA,
            pltpu.SemaphoreType.DMA,
        ],
    )(_gather_body)
    return k(A, B, srcg, dstg)


def _scatter_body(v_hbm, dsts, zeros_hbm, out_hbm,
                  idx_v, vb0, vb1, acc, s0, s1):
    c = lax.axis_index("c")
    s = lax.axis_index("s")
    w = s * _NC + c
    base = w * _EW
    pltpu.sync_copy(dsts.at[w], idx_v)
    vbufs = (vb0, vb1)
    sems = (s0, s1)

    for p in range(_NPASS):
        @pl.when(s < _NS - 1)
        def _zero_main():
            pltpu.sync_copy(zeros_hbm, acc.at[pl.ds(s * _NROW, _NROW)])

        @pl.when(s == _NS - 1)
        def _zero_last():
            pltpu.sync_copy(zeros_hbm.at[pl.ds(0, _NRL + 16)],
                            acc.at[pl.ds((_NS - 1) * _NROW, _NRL + 16)])

        plsc.subcore_barrier()

        pltpu.async_copy(v_hbm.at[pl.ds(base, _SCH), pl.ds(p * _HC, _HC)],
                         vbufs[0], sems[0])

        def outer(g):
            for par in range(2):
                j = g + par
                nb = 1 - par

                @pl.when(j + 1 < _SK)
                def _start_next():
                    pltpu.async_copy(
                        v_hbm.at[pl.ds(base + (j + 1) * _SCH, _SCH),
                                 pl.ds(p * _HC, _HC)],
                        vbufs[nb], sems[nb])

                pltpu.make_async_copy(
                    v_hbm.at[pl.ds(base + j * _SCH, _SCH), pl.ds(p * _HC, _HC)],
                    vbufs[par], sems[par]).wait()
                pltpu.sync_copy(vbufs[par], acc.at[idx_v.at[j]], add=True)

        pl.loop(0, _SK, step=2)(outer)
        plsc.subcore_barrier()

        @pl.when(s < _NS - 1)
        def _drain_main():
            pltpu.sync_copy(acc.at[pl.ds(s * _NROW, _NROW)],
                            out_hbm.at[c, pl.ds(s * _NROW, _NROW),
                                       pl.ds(p * _HC, _HC)])

        @pl.when(s == _NS - 1)
        def _drain_last():
            pltpu.sync_copy(acc.at[pl.ds((_NS - 1) * _NROW, _NRL)],
                            out_hbm.at[c, pl.ds((_NS - 1) * _NROW, _NRL),
                                       pl.ds(p * _HC, _HC)])

        plsc.subcore_barrier()


def _sc_scatter(v, dsts, zeros_hbm):
    k = functools.partial(
        pl.kernel,
        out_type=jax.ShapeDtypeStruct((_NC, _N, _H), jnp.float32),
        mesh=_sc_mesh(),
        scratch_types=[
            pltpu.VMEM((_SK, _SCH), jnp.int32),
            pltpu.VMEM((_SCH, _HC), jnp.float32),
            pltpu.VMEM((_SCH, _HC), jnp.float32),
            pltpu.VMEM_SHARED((_NACC, _HC), jnp.float32),
            pltpu.SemaphoreType.DMA,
            pltpu.SemaphoreType.DMA,
        ],
    )(_scatter_body)
    return k(v, dsts, zeros_hbm)


def kernel(x, edge_index, W_in, b_in,
           blk0_W1, blk0_b1, blk0_W2, blk0_b2, blk0_W3, blk0_b3,
           blk1_W1, blk1_b1, blk1_W2, blk1_b2, blk1_W3, blk1_b3,
           W_out, b_out):
    src = edge_index[0]
    dst = edge_index[1]
    pad = _EP - _E
    srcg = jnp.concatenate([src, jnp.zeros((pad,), jnp.int32)]).reshape(_NW, _GK, _GCH)
    dstg = jnp.concatenate([dst, jnp.zeros((pad,), jnp.int32)]).reshape(_NW, _GK, _GCH)
    dsts = jnp.concatenate([dst, jnp.full((pad,), _N, jnp.int32)]).reshape(_NW, _SK, _SCH)
    zeros_hbm = jnp.zeros((_NROW, _HC), jnp.float32)

    h0 = _mm_bias([x], W_in, b_in)

    parts = None
    for (W1, b1, W2, b2, W3, b3) in (
            (blk0_W1, blk0_b1, blk0_W2, blk0_b2, blk0_W3, blk0_b3),
            (blk1_W1, blk1_b1, blk1_W2, blk1_b2, blk1_W3, blk1_b3)):
        hin = [h0] if parts is None else [parts[0], parts[1]]
        A, B = _mm_ab(hin, W1, b1)
        aj, bi = _sc_gather(A, B, srcg, dstg)
        v = _edge_mlp(aj, bi, W2, b2, W3, b3)
        parts = _sc_scatter(v, dsts, zeros_hbm)

    return _mm_bias([parts[0], parts[1]], W_out, b_out)


# speedup vs baseline: 4.1920x; 1.8894x over previous
import functools

import jax
import jax.numpy as jnp
from jax import lax
from jax.experimental import pallas as pl
from jax.experimental.pallas import tpu as pltpu
from jax.experimental.pallas import tpu_sc as plsc

_N = 10000
_E = 160000
_H = 512
_NC = 2
_NS = 16
_NW = _NC * _NS
_EP = 163840
_EH = _EP // 2
_GCH = 64
_GPT = _EH // _GCH // _NS
_HH = _H // 2
_HQ = _HH // 2
_SCH = 128
_SK = 20
_EW = _SK * _SCH
_HC = 128
_NPASS = _H // _HC
_NROW = 632
_NRL = _N - 15 * _NROW
_NACC = _N + 16
_TE = 2048
_TM = 2000


def _sc_mesh():
    return plsc.VectorSubcoreMesh(core_axis_name="c", subcore_axis_name="s")


def _mm_body(*refs):
    n_parts = len(refs) - 3
    w, b, o = refs[n_parts], refs[n_parts + 1], refs[n_parts + 2]
    h = refs[0][...]
    for r in refs[1:n_parts]:
        h = h + r[...]
    o[...] = jnp.dot(h, w[...], preferred_element_type=jnp.float32) + b[...]


def _mm_bias(parts, W, bvec):
    M, K = parts[0].shape
    Nc = W.shape[1]
    part_specs = [pl.BlockSpec((_TM, K), lambda i: (i, 0)) for _ in parts]
    return pl.pallas_call(
        _mm_body,
        grid=(M // _TM,),
        in_specs=part_specs + [
            pl.BlockSpec((K, Nc), lambda i: (0, 0)),
            pl.BlockSpec((1, Nc), lambda i: (0, 0)),
        ],
        out_specs=pl.BlockSpec((_TM, Nc), lambda i: (i, 0)),
        out_shape=jax.ShapeDtypeStruct((M, Nc), jnp.float32),
    )(*parts, W, bvec.reshape(1, Nc))


def _pack_bf16(x32):
    half = x32.shape[1] // 2
    u = jax.lax.bitcast_convert_type(x32.astype(jnp.bfloat16),
                                     jnp.uint16).astype(jnp.uint32)
    w = u[:, :half] | (u[:, half:] << 16)
    return jax.lax.bitcast_convert_type(w, jnp.int32)


def _unpack_bf16(p):
    u = jax.lax.bitcast_convert_type(p, jnp.uint32)
    lo = jax.lax.bitcast_convert_type((u & 0xFFFF).astype(jnp.uint16),
                                      jnp.bfloat16)
    hi = jax.lax.bitcast_convert_type((u >> 16).astype(jnp.uint16),
                                      jnp.bfloat16)
    return lo, hi


def _ab_body(*refs):
    n_parts = len(refs) - 5
    wt, wb, b1, a, b = refs[n_parts:]
    h = refs[0][...]
    for r in refs[1:n_parts]:
        h = h + r[...]
    a[...] = _pack_bf16(jnp.dot(h, wt[...],
                                preferred_element_type=jnp.float32) + b1[...])
    b[...] = _pack_bf16(jnp.dot(h, wb[...],
                                preferred_element_type=jnp.float32))


def _mm_ab(parts, W1, b1):
    M, K = parts[0].shape
    part_specs = [pl.BlockSpec((_TM, K), lambda i: (i, 0)) for _ in parts]
    out_sds = jax.ShapeDtypeStruct((M, _HH), jnp.int32)
    return pl.pallas_call(
        _ab_body,
        grid=(M // _TM,),
        in_specs=part_specs + [
            pl.BlockSpec((K, _H), lambda i: (0, 0)),
            pl.BlockSpec((K, _H), lambda i: (0, 0)),
            pl.BlockSpec((1, _H), lambda i: (0, 0)),
        ],
        out_specs=[pl.BlockSpec((_TM, _HH), lambda i: (i, 0)),
                   pl.BlockSpec((_TM, _HH), lambda i: (i, 0))],
        out_shape=[out_sds, out_sds],
    )(*parts, W1[:_H], W1[_H:], b1.reshape(1, _H))


def _mlp_body(aj, bi, w2, b2, w3, b3, v):
    aj_lo, aj_hi = _unpack_bf16(aj[...])
    bi_lo, bi_hi = _unpack_bf16(bi[...])
    t_lo = jnp.maximum(aj_lo + bi_lo, jnp.bfloat16(0.0))
    t_hi = jnp.maximum(aj_hi + bi_hi, jnp.bfloat16(0.0))
    w2v = w2[...]
    u = (jnp.dot(t_lo, w2v[:_H // 2], preferred_element_type=jnp.float32)
         + jnp.dot(t_hi, w2v[_H // 2:], preferred_element_type=jnp.float32)
         + b2[...])
    u = jnp.maximum(u, 0.0).astype(jnp.bfloat16)
    v[...] = jnp.dot(u, w3[...], preferred_element_type=jnp.float32) + b3[...]


def _edge_mlp(aj, bi, W2, b2, W3, b3):
    return pl.pallas_call(
        _mlp_body,
        grid=(_EH // _TE,),
        in_specs=[
            pl.BlockSpec((_TE, _HH), lambda i: (i, 0)),
            pl.BlockSpec((_TE, _HH), lambda i: (i, 0)),
            pl.BlockSpec((_H, _H), lambda i: (0, 0)),
            pl.BlockSpec((1, _H), lambda i: (0, 0)),
            pl.BlockSpec((_H, _H), lambda i: (0, 0)),
            pl.BlockSpec((1, _H), lambda i: (0, 0)),
        ],
        out_specs=pl.BlockSpec((_TE, _H), lambda i: (i, 0)),
        out_shape=jax.ShapeDtypeStruct((_EH, _H), jnp.float32),
    )(aj, bi, W2.astype(jnp.bfloat16), b2.reshape(1, _H),
      W3.astype(jnp.bfloat16), b3.reshape(1, _H))


def _gather_body(a_hbm, b_hbm, srcg, dstg, aj_hbm, bi_hbm,
                 idx_v, r0, r1, tbl, s0, s1):
    c = lax.axis_index("c")
    s = lax.axis_index("s")
    bufs = (r0, r1)
    sems = (s0, s1)

    def side(table_hbm, idx_src, out):
        pltpu.sync_copy(idx_src.at[pl.ds(s * _GPT, _GPT)], idx_v)
        for p in range(2):
            @pl.when(s < _NS - 1)
            def _load_main():
                pltpu.sync_copy(
                    table_hbm.at[pl.ds(s * _NROW, _NROW), pl.ds(p * _HQ, _HQ)],
                    tbl.at[pl.ds(s * _NROW, _NROW)])

            @pl.when(s == _NS - 1)
            def _load_last():
                pltpu.sync_copy(
                    table_hbm.at[pl.ds((_NS - 1) * _NROW, _NRL),
                                 pl.ds(p * _HQ, _HQ)],
                    tbl.at[pl.ds((_NS - 1) * _NROW, _NRL)])

            plsc.subcore_barrier()
            pltpu.async_copy(tbl.at[idx_v.at[0]], bufs[0], sems[0])

            def outer(g):
                for par in range(2):
                    j = g + par
                    nb = 1 - par

                    @pl.when(j + 1 < _GPT)
                    def _start_next():
                        pltpu.async_copy(tbl.at[idx_v.at[j + 1]],
                                         bufs[nb], sems[nb])

                    pltpu.make_async_copy(tbl.at[idx_v.at[j]],
                                          bufs[par], sems[par]).wait()
                    pltpu.sync_copy(
                        bufs[par],
                        out.at[pl.ds((s * _GPT + j) * _GCH, _GCH),
                               pl.ds(p * _HQ, _HQ)])

            pl.loop(0, _GPT, step=2)(outer)
            plsc.subcore_barrier()

    @pl.when(c == 0)
    def _side_a():
        side(a_hbm, srcg, aj_hbm)

    @pl.when(c == 1)
    def _side_b():
        side(b_hbm, dstg, bi_hbm)


def _sc_gather(A, B, srcg, dstg):
    out_sds = jax.ShapeDtypeStruct((_EH, _HH), jnp.int32)
    k = functools.partial(
        pl.kernel,
        out_type=(out_sds, out_sds),
        mesh=_sc_mesh(),
        scratch_types=[
            pltpu.VMEM((_GPT, _GCH), jnp.int32),
            pltpu.VMEM((_GCH, _HQ), jnp.int32),
            pltpu.VMEM((_GCH, _HQ), jnp.int32),
            pltpu.VMEM_SHARED((_N, _HQ), jnp.int32),
            pltpu.SemaphoreType.DMA,
            pltpu.SemaphoreType.DMA,
        ],
    )(_gather_body)
    return k(A, B, srcg, dstg)


def _scatter_body(chained, *args):
    if chained:
        (v_hbm, dsts, zeros_hbm, init0, init1, out0, out1,
         idx_v, vb0, vb1, acc, s0, s1) = args
        init = (init0, init1)
    else:
        (v_hbm, dsts, zeros_hbm, out0, out1,
         idx_v, vb0, vb1, acc, s0, s1) = args
    c = lax.axis_index("c")
    s = lax.axis_index("s")
    w = s * _NC + c
    base = w * _EW
    pltpu.sync_copy(dsts.at[w], idx_v)
    vbufs = (vb0, vb1)
    sems = (s0, s1)
    out = (out0, out1)

    for p in range(_NPASS):
        if chained:
            for q in range(_NC):
                @pl.when(c == q)
                def _init_q(q=q):
                    @pl.when(s < _NS - 1)
                    def _init_main():
                        pltpu.sync_copy(
                            init[q].at[pl.ds(s * _NROW, _NROW),
                                       pl.ds(p * _HC, _HC)],
                            acc.at[pl.ds(s * _NROW, _NROW)])

                    @pl.when(s == _NS - 1)
                    def _init_last():
                        pltpu.sync_copy(
                            init[q].at[pl.ds((_NS - 1) * _NROW, _NRL),
                                       pl.ds(p * _HC, _HC)],
                            acc.at[pl.ds((_NS - 1) * _NROW, _NRL)])
        else:
            @pl.when(s < _NS - 1)
            def _zero_main():
                pltpu.sync_copy(zeros_hbm, acc.at[pl.ds(s * _NROW, _NROW)])

            @pl.when(s == _NS - 1)
            def _zero_last():
                pltpu.sync_copy(zeros_hbm.at[pl.ds(0, _NRL + 16)],
                                acc.at[pl.ds((_NS - 1) * _NROW, _NRL + 16)])

        plsc.subcore_barrier()

        pltpu.async_copy(v_hbm.at[pl.ds(base, _SCH), pl.ds(p * _HC, _HC)],
                         vbufs[0], sems[0])

        def outer(g):
            for par in range(2):
                j = g + par
                nb = 1 - par

                @pl.when(j + 1 < _SK)
                def _start_next():
                    pltpu.async_copy(
                        v_hbm.at[pl.ds(base + (j + 1) * _SCH, _SCH),
                                 pl.ds(p * _HC, _HC)],
                        vbufs[nb], sems[nb])

                pltpu.make_async_copy(
                    v_hbm.at[pl.ds(base + j * _SCH, _SCH), pl.ds(p * _HC, _HC)],
                    vbufs[par], sems[par]).wait()
                pltpu.sync_copy(vbufs[par], acc.at[idx_v.at[j]], add=True)

        pl.loop(0, _SK, step=2)(outer)
        plsc.subcore_barrier()

        for q in range(_NC):
            @pl.when(c == q)
            def _drain_q(q=q):
                @pl.when(s < _NS - 1)
                def _drain_main():
                    pltpu.sync_copy(
                        acc.at[pl.ds(s * _NROW, _NROW)],
                        out[q].at[pl.ds(s * _NROW, _NROW),
                                  pl.ds(p * _HC, _HC)])

                @pl.when(s == _NS - 1)
                def _drain_last():
                    pltpu.sync_copy(
                        acc.at[pl.ds((_NS - 1) * _NROW, _NRL)],
                        out[q].at[pl.ds((_NS - 1) * _NROW, _NRL),
                                  pl.ds(p * _HC, _HC)])

        plsc.subcore_barrier()


def _sc_scatter(v, dsts, zeros_hbm, inits=None):
    out_sds = jax.ShapeDtypeStruct((_N, _H), jnp.float32)
    chained = inits is not None
    k = functools.partial(
        pl.kernel,
        out_type=(out_sds, out_sds),
        mesh=_sc_mesh(),
        scratch_types=[
            pltpu.VMEM((_SK, _SCH), jnp.int32),
            pltpu.VMEM((_SCH, _HC), jnp.float32),
            pltpu.VMEM((_SCH, _HC), jnp.float32),
            pltpu.VMEM_SHARED((_NACC, _HC), jnp.float32),
            pltpu.SemaphoreType.DMA,
            pltpu.SemaphoreType.DMA,
        ],
    )(functools.partial(_scatter_body, chained))
    if chained:
        return k(v, dsts, zeros_hbm, inits[0], inits[1])
    return k(v, dsts, zeros_hbm)


def kernel(x, edge_index, W_in, b_in,
           blk0_W1, blk0_b1, blk0_W2, blk0_b2, blk0_W3, blk0_b3,
           blk1_W1, blk1_b1, blk1_W2, blk1_b2, blk1_W3, blk1_b3,
           W_out, b_out):
    src = edge_index[0]
    dst = edge_index[1]
    pad = _EP - _E
    srcg = jnp.concatenate(
        [src, jnp.zeros((pad,), jnp.int32)]).reshape(2, _EH // _GCH, _GCH)
    dstg = jnp.concatenate(
        [dst, jnp.zeros((pad,), jnp.int32)]).reshape(2, _EH // _GCH, _GCH)
    dsts = jnp.concatenate(
        [dst, jnp.full((pad,), _N, jnp.int32)]).reshape(2, _NW, _SK, _SCH)
    zeros_hbm = jnp.zeros((_NROW, _HC), jnp.float32)

    h0 = _mm_bias([x], W_in, b_in)

    parts = None
    for (W1, b1, W2, b2, W3, b3) in (
            (blk0_W1, blk0_b1, blk0_W2, blk0_b2, blk0_W3, blk0_b3),
            (blk1_W1, blk1_b1, blk1_W2, blk1_b2, blk1_W3, blk1_b3)):
        hin = [h0] if parts is None else list(parts)
        A, B = _mm_ab(hin, W1, b1)
        aj0, bi0 = _sc_gather(A, B, srcg[0], dstg[0])
        v0 = _edge_mlp(aj0, bi0, W2, b2, W3, b3)
        aj1, bi1 = _sc_gather(A, B, srcg[1], dstg[1])
        v1 = _edge_mlp(aj1, bi1, W2, b2, W3, b3)
        p0 = _sc_scatter(v0, dsts[0], zeros_hbm)
        parts = _sc_scatter(v1, dsts[1], zeros_hbm, inits=p0)

    return _mm_bias(list(parts), W_out, b_out)


# speedup vs baseline: 4.6436x; 1.1077x over previous
import functools

import jax
import jax.numpy as jnp
from jax import lax
from jax.experimental import pallas as pl
from jax.experimental.pallas import tpu as pltpu
from jax.experimental.pallas import tpu_sc as plsc

_N = 10000
_E = 160000
_H = 512
_NC = 2
_NS = 16
_NW = _NC * _NS
_EP = 163840
_EH = _EP // 2
_GCH = 64
_GPT = _EH // _GCH // _NS
_HH = _H // 2
_HQ = _HH // 2
_SCH = 128
_SK = 40
_EW = _SK * _SCH
_CPC = _H // _NC
_HC = 128
_NPASS = _H // _HC
_NROW = 632
_NRL = _N - 15 * _NROW
_NACC = _N + 16
_TE = 2048
_TM = 2000


def _sc_mesh():
    return plsc.VectorSubcoreMesh(core_axis_name="c", subcore_axis_name="s")


def _mm_body(*refs):
    n_parts = len(refs) - 3
    w, b, o = refs[n_parts], refs[n_parts + 1], refs[n_parts + 2]
    h = refs[0][...]
    for r in refs[1:n_parts]:
        h = h + r[...]
    o[...] = jnp.dot(h, w[...], preferred_element_type=jnp.float32) + b[...]


def _mm_bias(parts, W, bvec):
    M, K = parts[0].shape
    Nc = W.shape[1]
    part_specs = [pl.BlockSpec((_TM, K), lambda i: (i, 0)) for _ in parts]
    return pl.pallas_call(
        _mm_body,
        grid=(M // _TM,),
        in_specs=part_specs + [
            pl.BlockSpec((K, Nc), lambda i: (0, 0)),
            pl.BlockSpec((1, Nc), lambda i: (0, 0)),
        ],
        out_specs=pl.BlockSpec((_TM, Nc), lambda i: (i, 0)),
        out_shape=jax.ShapeDtypeStruct((M, Nc), jnp.float32),
    )(*parts, W, bvec.reshape(1, Nc))


def _pack_bf16(x32):
    half = x32.shape[1] // 2
    u = jax.lax.bitcast_convert_type(x32.astype(jnp.bfloat16),
                                     jnp.uint16).astype(jnp.uint32)
    w = u[:, :half] | (u[:, half:] << 16)
    return jax.lax.bitcast_convert_type(w, jnp.int32)


def _unpack_bf16(p):
    u = jax.lax.bitcast_convert_type(p, jnp.uint32)
    lo = jax.lax.bitcast_convert_type((u & 0xFFFF).astype(jnp.uint16),
                                      jnp.bfloat16)
    hi = jax.lax.bitcast_convert_type((u >> 16).astype(jnp.uint16),
                                      jnp.bfloat16)
    return lo, hi


def _ab_body(*refs):
    n_parts = len(refs) - 5
    wt, wb, b1, a, b = refs[n_parts:]
    h = refs[0][...]
    for r in refs[1:n_parts]:
        h = h + r[...]
    a[...] = _pack_bf16(jnp.dot(h, wt[...],
                                preferred_element_type=jnp.float32) + b1[...])
    b[...] = _pack_bf16(jnp.dot(h, wb[...],
                                preferred_element_type=jnp.float32))


def _mm_ab(parts, W1, b1):
    M, K = parts[0].shape
    part_specs = [pl.BlockSpec((_TM, K), lambda i: (i, 0)) for _ in parts]
    out_sds = jax.ShapeDtypeStruct((M, _HH), jnp.int32)
    return pl.pallas_call(
        _ab_body,
        grid=(M // _TM,),
        in_specs=part_specs + [
            pl.BlockSpec((K, _H), lambda i: (0, 0)),
            pl.BlockSpec((K, _H), lambda i: (0, 0)),
            pl.BlockSpec((1, _H), lambda i: (0, 0)),
        ],
        out_specs=[pl.BlockSpec((_TM, _HH), lambda i: (i, 0)),
                   pl.BlockSpec((_TM, _HH), lambda i: (i, 0))],
        out_shape=[out_sds, out_sds],
    )(*parts, W1[:_H], W1[_H:], b1.reshape(1, _H))


def _mlp_body(aj, bi, w2, b2, w3, b3, v):
    aj_lo, aj_hi = _unpack_bf16(aj[...])
    bi_lo, bi_hi = _unpack_bf16(bi[...])
    t_lo = jnp.maximum(aj_lo + bi_lo, jnp.bfloat16(0.0))
    t_hi = jnp.maximum(aj_hi + bi_hi, jnp.bfloat16(0.0))
    w2v = w2[...]
    u = (jnp.dot(t_lo, w2v[:_H // 2], preferred_element_type=jnp.float32)
         + jnp.dot(t_hi, w2v[_H // 2:], preferred_element_type=jnp.float32)
         + b2[...])
    u = jnp.maximum(u, 0.0).astype(jnp.bfloat16)
    v[...] = jnp.dot(u, w3[...], preferred_element_type=jnp.float32) + b3[...]


def _edge_mlp(aj, bi, W2, b2, W3, b3):
    return pl.pallas_call(
        _mlp_body,
        grid=(_EH // _TE,),
        in_specs=[
            pl.BlockSpec((_TE, _HH), lambda i: (i, 0)),
            pl.BlockSpec((_TE, _HH), lambda i: (i, 0)),
            pl.BlockSpec((_H, _H), lambda i: (0, 0)),
            pl.BlockSpec((1, _H), lambda i: (0, 0)),
            pl.BlockSpec((_H, _H), lambda i: (0, 0)),
            pl.BlockSpec((1, _H), lambda i: (0, 0)),
        ],
        out_specs=pl.BlockSpec((_TE, _H), lambda i: (i, 0)),
        out_shape=jax.ShapeDtypeStruct((_EH, _H), jnp.float32),
    )(aj, bi, W2.astype(jnp.bfloat16), b2.reshape(1, _H),
      W3.astype(jnp.bfloat16), b3.reshape(1, _H))


def _gather_body(a_hbm, b_hbm, srcg, dstg, aj_hbm, bi_hbm,
                 idx_v, r0, r1, tbl, s0, s1):
    c = lax.axis_index("c")
    s = lax.axis_index("s")
    bufs = (r0, r1)
    sems = (s0, s1)

    def side(table_hbm, idx_src, out):
        pltpu.sync_copy(idx_src.at[pl.ds(s * _GPT, _GPT)], idx_v)
        for p in range(2):
            @pl.when(s < _NS - 1)
            def _load_main():
                pltpu.sync_copy(
                    table_hbm.at[pl.ds(s * _NROW, _NROW), pl.ds(p * _HQ, _HQ)],
                    tbl.at[pl.ds(s * _NROW, _NROW)])

            @pl.when(s == _NS - 1)
            def _load_last():
                pltpu.sync_copy(
                    table_hbm.at[pl.ds((_NS - 1) * _NROW, _NRL),
                                 pl.ds(p * _HQ, _HQ)],
                    tbl.at[pl.ds((_NS - 1) * _NROW, _NRL)])

            plsc.subcore_barrier()
            pltpu.async_copy(tbl.at[idx_v.at[0]], bufs[0], sems[0])

            def outer(g):
                for par in range(2):
                    j = g + par
                    nb = 1 - par

                    @pl.when(j + 1 < _GPT)
                    def _start_next():
                        pltpu.async_copy(tbl.at[idx_v.at[j + 1]],
                                         bufs[nb], sems[nb])

                    pltpu.make_async_copy(tbl.at[idx_v.at[j]],
                                          bufs[par], sems[par]).wait()
                    pltpu.sync_copy(
                        bufs[par],
                        out.at[pl.ds((s * _GPT + j) * _GCH, _GCH),
                               pl.ds(p * _HQ, _HQ)])

            pl.loop(0, _GPT, step=2)(outer)
            plsc.subcore_barrier()

    @pl.when(c == 0)
    def _side_a():
        side(a_hbm, srcg, aj_hbm)

    @pl.when(c == 1)
    def _side_b():
        side(b_hbm, dstg, bi_hbm)


def _sc_gather(A, B, srcg, dstg):
    out_sds = jax.ShapeDtypeStruct((_EH, _HH), jnp.int32)
    k = functools.partial(
        pl.kernel,
        out_type=(out_sds, out_sds),
        mesh=_sc_mesh(),
        scratch_types=[
            pltpu.VMEM((_GPT, _GCH), jnp.int32),
            pltpu.VMEM((_GCH, _HQ), jnp.int32),
            pltpu.VMEM((_GCH, _HQ), jnp.int32),
            pltpu.VMEM_SHARED((_N, _HQ), jnp.int32),
            pltpu.SemaphoreType.DMA,
            pltpu.SemaphoreType.DMA,
        ],
    )(_gather_body)
    return k(A, B, srcg, dstg)


def _scatter_body(chained, *args):
    if chained:
        (v_hbm, dsts, zeros_hbm, init, out,
         idx_v, vb0, vb1, acc, s0, s1) = args
    else:
        (v_hbm, dsts, zeros_hbm, out,
         idx_v, vb0, vb1, acc, s0, s1) = args
    c = lax.axis_index("c")
    s = lax.axis_index("s")
    base = s * _EW
    pltpu.sync_copy(dsts.at[s], idx_v)
    vbufs = (vb0, vb1)
    sems = (s0, s1)

    for p in range(_H // _CPC):
        col = c * _CPC + p * _HC
        if chained:
            @pl.when(s < _NS - 1)
            def _init_main():
                pltpu.sync_copy(
                    init.at[pl.ds(s * _NROW, _NROW), pl.ds(col, _HC)],
                    acc.at[pl.ds(s * _NROW, _NROW)])

            @pl.when(s == _NS - 1)
            def _init_last():
                pltpu.sync_copy(
                    init.at[pl.ds((_NS - 1) * _NROW, _NRL), pl.ds(col, _HC)],
                    acc.at[pl.ds((_NS - 1) * _NROW, _NRL)])
        else:
            @pl.when(s < _NS - 1)
            def _zero_main():
                pltpu.sync_copy(zeros_hbm, acc.at[pl.ds(s * _NROW, _NROW)])

            @pl.when(s == _NS - 1)
            def _zero_last():
                pltpu.sync_copy(zeros_hbm.at[pl.ds(0, _NRL)],
                                acc.at[pl.ds((_NS - 1) * _NROW, _NRL)])

        plsc.subcore_barrier()

        pltpu.async_copy(v_hbm.at[pl.ds(base, _SCH), pl.ds(col, _HC)],
                         vbufs[0], sems[0])

        def outer(g):
            for par in range(2):
                j = g + par
                nb = 1 - par

                @pl.when(j + 1 < _SK)
                def _start_next():
                    pltpu.async_copy(
                        v_hbm.at[pl.ds(base + (j + 1) * _SCH, _SCH),
                                 pl.ds(col, _HC)],
                        vbufs[nb], sems[nb])

                pltpu.make_async_copy(
                    v_hbm.at[pl.ds(base + j * _SCH, _SCH), pl.ds(col, _HC)],
                    vbufs[par], sems[par]).wait()
                pltpu.sync_copy(vbufs[par], acc.at[idx_v.at[j]], add=True)

        pl.loop(0, _SK, step=2)(outer)
        plsc.subcore_barrier()

        @pl.when(s < _NS - 1)
        def _drain_main():
            pltpu.sync_copy(acc.at[pl.ds(s * _NROW, _NROW)],
                            out.at[pl.ds(s * _NROW, _NROW), pl.ds(col, _HC)])

        @pl.when(s == _NS - 1)
        def _drain_last():
            pltpu.sync_copy(
                acc.at[pl.ds((_NS - 1) * _NROW, _NRL)],
                out.at[pl.ds((_NS - 1) * _NROW, _NRL), pl.ds(col, _HC)])

        plsc.subcore_barrier()


def _sc_scatter(v, dsts, zeros_hbm, init=None):
    out_sds = jax.ShapeDtypeStruct((_N, _H), jnp.float32)
    chained = init is not None
    k = functools.partial(
        pl.kernel,
        out_type=out_sds,
        mesh=_sc_mesh(),
        scratch_types=[
            pltpu.VMEM((_SK, _SCH), jnp.int32),
            pltpu.VMEM((_SCH, _HC), jnp.float32),
            pltpu.VMEM((_SCH, _HC), jnp.float32),
            pltpu.VMEM_SHARED((_NACC, _HC), jnp.float32),
            pltpu.SemaphoreType.DMA,
            pltpu.SemaphoreType.DMA,
        ],
    )(functools.partial(_scatter_body, chained))
    if chained:
        return k(v, dsts, zeros_hbm, init)
    return k(v, dsts, zeros_hbm)


def kernel(x, edge_index, W_in, b_in,
           blk0_W1, blk0_b1, blk0_W2, blk0_b2, blk0_W3, blk0_b3,
           blk1_W1, blk1_b1, blk1_W2, blk1_b2, blk1_W3, blk1_b3,
           W_out, b_out):
    src = edge_index[0]
    dst = edge_index[1]
    pad = _EP - _E
    srcg = jnp.concatenate(
        [src, jnp.zeros((pad,), jnp.int32)]).reshape(2, _EH // _GCH, _GCH)
    dstg = jnp.concatenate(
        [dst, jnp.zeros((pad,), jnp.int32)]).reshape(2, _EH // _GCH, _GCH)
    dsts = jnp.concatenate(
        [dst, jnp.full((pad,), _N, jnp.int32)]).reshape(2, _NS, _SK, _SCH)
    zeros_hbm = jnp.zeros((_NROW, _HC), jnp.float32)

    h0 = _mm_bias([x], W_in, b_in)

    parts = None
    for (W1, b1, W2, b2, W3, b3) in (
            (blk0_W1, blk0_b1, blk0_W2, blk0_b2, blk0_W3, blk0_b3),
            (blk1_W1, blk1_b1, blk1_W2, blk1_b2, blk1_W3, blk1_b3)):
        hin = [h0] if parts is None else list(parts)
        A, B = _mm_ab(hin, W1, b1)
        aj0, bi0 = _sc_gather(A, B, srcg[0], dstg[0])
        v0 = _edge_mlp(aj0, bi0, W2, b2, W3, b3)
        aj1, bi1 = _sc_gather(A, B, srcg[1], dstg[1])
        v1 = _edge_mlp(aj1, bi1, W2, b2, W3, b3)
        p0 = _sc_scatter(v0, dsts[0], zeros_hbm)
        parts = [_sc_scatter(v1, dsts[1], zeros_hbm, init=p0)]

    return _mm_bias(parts, W_out, b_out)


# speedup vs baseline: 4.6813x; 1.0081x over previous
import functools

import jax
import jax.numpy as jnp
from jax import lax
from jax.experimental import pallas as pl
from jax.experimental.pallas import tpu as pltpu
from jax.experimental.pallas import tpu_sc as plsc

_N = 10000
_E = 160000
_H = 512
_NC = 2
_NS = 16
_NW = _NC * _NS
_EP = 163840
_EH = _EP // 2
_GCH = 64
_GPT = _EH // _GCH // _NS
_HH = _H // 2
_HQ = _HH // 2
_SCH = 128
_SK = 40
_EW = _SK * _SCH
_CPC = _H // _NC
_HC = 128
_NPASS = _H // _HC
_NROW = 632
_NRL = _N - 15 * _NROW
_NACC = _N + 16
_TE = 4096
_TM = 2000


def _sc_mesh():
    return plsc.VectorSubcoreMesh(core_axis_name="c", subcore_axis_name="s")


def _mm_body(*refs):
    n_parts = len(refs) - 3
    w, b, o = refs[n_parts], refs[n_parts + 1], refs[n_parts + 2]
    h = refs[0][...]
    for r in refs[1:n_parts]:
        h = h + r[...]
    o[...] = jnp.dot(h, w[...], preferred_element_type=jnp.float32) + b[...]


def _mm_bias(parts, W, bvec):
    M, K = parts[0].shape
    Nc = W.shape[1]
    part_specs = [pl.BlockSpec((_TM, K), lambda i: (i, 0)) for _ in parts]
    return pl.pallas_call(
        _mm_body,
        grid=(M // _TM,),
        in_specs=part_specs + [
            pl.BlockSpec((K, Nc), lambda i: (0, 0)),
            pl.BlockSpec((1, Nc), lambda i: (0, 0)),
        ],
        out_specs=pl.BlockSpec((_TM, Nc), lambda i: (i, 0)),
        out_shape=jax.ShapeDtypeStruct((M, Nc), jnp.float32),
    )(*parts, W, bvec.reshape(1, Nc))


def _pack_bf16(x32):
    half = x32.shape[1] // 2
    u = jax.lax.bitcast_convert_type(x32.astype(jnp.bfloat16),
                                     jnp.uint16).astype(jnp.uint32)
    w = u[:, :half] | (u[:, half:] << 16)
    return jax.lax.bitcast_convert_type(w, jnp.int32)


def _unpack_bf16(p):
    u = jax.lax.bitcast_convert_type(p, jnp.uint32)
    lo = jax.lax.bitcast_convert_type((u & 0xFFFF).astype(jnp.uint16),
                                      jnp.bfloat16)
    hi = jax.lax.bitcast_convert_type((u >> 16).astype(jnp.uint16),
                                      jnp.bfloat16)
    return lo, hi


def _ab_body(*refs):
    n_parts = len(refs) - 5
    wt, wb, b1, a, b = refs[n_parts:]
    h = refs[0][...]
    for r in refs[1:n_parts]:
        h = h + r[...]
    a[...] = _pack_bf16(jnp.dot(h, wt[...],
                                preferred_element_type=jnp.float32) + b1[...])
    b[...] = _pack_bf16(jnp.dot(h, wb[...],
                                preferred_element_type=jnp.float32))


def _mm_ab(parts, W1, b1):
    M, K = parts[0].shape
    part_specs = [pl.BlockSpec((_TM, K), lambda i: (i, 0)) for _ in parts]
    out_sds = jax.ShapeDtypeStruct((M, _HH), jnp.int32)
    return pl.pallas_call(
        _ab_body,
        grid=(M // _TM,),
        in_specs=part_specs + [
            pl.BlockSpec((K, _H), lambda i: (0, 0)),
            pl.BlockSpec((K, _H), lambda i: (0, 0)),
            pl.BlockSpec((1, _H), lambda i: (0, 0)),
        ],
        out_specs=[pl.BlockSpec((_TM, _HH), lambda i: (i, 0)),
                   pl.BlockSpec((_TM, _HH), lambda i: (i, 0))],
        out_shape=[out_sds, out_sds],
    )(*parts, W1[:_H], W1[_H:], b1.reshape(1, _H))


def _mlp_body(aj, bi, w2, b2, w3, b3, v):
    aj_lo, aj_hi = _unpack_bf16(aj[...])
    bi_lo, bi_hi = _unpack_bf16(bi[...])
    t_lo = jnp.maximum(aj_lo + bi_lo, jnp.bfloat16(0.0))
    t_hi = jnp.maximum(aj_hi + bi_hi, jnp.bfloat16(0.0))
    w2v = w2[...]
    u = (jnp.dot(t_lo, w2v[:_H // 2], preferred_element_type=jnp.float32)
         + jnp.dot(t_hi, w2v[_H // 2:], preferred_element_type=jnp.float32)
         + b2[...])
    u = jnp.maximum(u, 0.0).astype(jnp.bfloat16)
    v[...] = jnp.dot(u, w3[...], preferred_element_type=jnp.float32) + b3[...]


def _edge_mlp(aj, bi, W2, b2, W3, b3):
    return pl.pallas_call(
        _mlp_body,
        grid=(_EH // _TE,),
        in_specs=[
            pl.BlockSpec((_TE, _HH), lambda i: (i, 0)),
            pl.BlockSpec((_TE, _HH), lambda i: (i, 0)),
            pl.BlockSpec((_H, _H), lambda i: (0, 0)),
            pl.BlockSpec((1, _H), lambda i: (0, 0)),
            pl.BlockSpec((_H, _H), lambda i: (0, 0)),
            pl.BlockSpec((1, _H), lambda i: (0, 0)),
        ],
        out_specs=pl.BlockSpec((_TE, _H), lambda i: (i, 0)),
        out_shape=jax.ShapeDtypeStruct((_EH, _H), jnp.float32),
    )(aj, bi, W2.astype(jnp.bfloat16), b2.reshape(1, _H),
      W3.astype(jnp.bfloat16), b3.reshape(1, _H))


def _gather_body(a_hbm, b_hbm, srcg, dstg, aj_hbm, bi_hbm,
                 idx_v, r0, r1, tbl, s0, s1):
    c = lax.axis_index("c")
    s = lax.axis_index("s")
    bufs = (r0, r1)
    sems = (s0, s1)

    def side(table_hbm, idx_src, out):
        pltpu.sync_copy(idx_src.at[pl.ds(s * _GPT, _GPT)], idx_v)
        for p in range(2):
            @pl.when(s < _NS - 1)
            def _load_main():
                pltpu.sync_copy(
                    table_hbm.at[pl.ds(s * _NROW, _NROW), pl.ds(p * _HQ, _HQ)],
                    tbl.at[pl.ds(s * _NROW, _NROW)])

            @pl.when(s == _NS - 1)
            def _load_last():
                pltpu.sync_copy(
                    table_hbm.at[pl.ds((_NS - 1) * _NROW, _NRL),
                                 pl.ds(p * _HQ, _HQ)],
                    tbl.at[pl.ds((_NS - 1) * _NROW, _NRL)])

            plsc.subcore_barrier()
            pltpu.async_copy(tbl.at[idx_v.at[0]], bufs[0], sems[0])

            def outer(g):
                for par in range(2):
                    j = g + par
                    nb = 1 - par

                    @pl.when(j + 1 < _GPT)
                    def _start_next():
                        pltpu.async_copy(tbl.at[idx_v.at[j + 1]],
                                         bufs[nb], sems[nb])

                    pltpu.make_async_copy(tbl.at[idx_v.at[j]],
                                          bufs[par], sems[par]).wait()
                    pltpu.sync_copy(
                        bufs[par],
                        out.at[pl.ds((s * _GPT + j) * _GCH, _GCH),
                               pl.ds(p * _HQ, _HQ)])

            pl.loop(0, _GPT, step=2)(outer)
            plsc.subcore_barrier()

    @pl.when(c == 0)
    def _side_a():
        side(a_hbm, srcg, aj_hbm)

    @pl.when(c == 1)
    def _side_b():
        side(b_hbm, dstg, bi_hbm)


def _sc_gather(A, B, srcg, dstg):
    out_sds = jax.ShapeDtypeStruct((_EH, _HH), jnp.int32)
    k = functools.partial(
        pl.kernel,
        out_type=(out_sds, out_sds),
        mesh=_sc_mesh(),
        scratch_types=[
            pltpu.VMEM((_GPT, _GCH), jnp.int32),
            pltpu.VMEM((_GCH, _HQ), jnp.int32),
            pltpu.VMEM((_GCH, _HQ), jnp.int32),
            pltpu.VMEM_SHARED((_N, _HQ), jnp.int32),
            pltpu.SemaphoreType.DMA,
            pltpu.SemaphoreType.DMA,
        ],
    )(_gather_body)
    return k(A, B, srcg, dstg)


def _scatter_body(chained, *args):
    if chained:
        (v_hbm, dsts, zeros_hbm, init, out,
         idx_v, vb0, vb1, acc, s0, s1) = args
    else:
        (v_hbm, dsts, zeros_hbm, out,
         idx_v, vb0, vb1, acc, s0, s1) = args
    c = lax.axis_index("c")
    s = lax.axis_index("s")
    base = s * _EW
    pltpu.sync_copy(dsts.at[s], idx_v)
    vbufs = (vb0, vb1)
    sems = (s0, s1)

    for p in range(_H // _CPC):
        col = c * _CPC + p * _HC
        if chained:
            @pl.when(s < _NS - 1)
            def _init_main():
                pltpu.sync_copy(
                    init.at[pl.ds(s * _NROW, _NROW), pl.ds(col, _HC)],
                    acc.at[pl.ds(s * _NROW, _NROW)])

            @pl.when(s == _NS - 1)
            def _init_last():
                pltpu.sync_copy(
                    init.at[pl.ds((_NS - 1) * _NROW, _NRL), pl.ds(col, _HC)],
                    acc.at[pl.ds((_NS - 1) * _NROW, _NRL)])
        else:
            @pl.when(s < _NS - 1)
            def _zero_main():
                pltpu.sync_copy(zeros_hbm, acc.at[pl.ds(s * _NROW, _NROW)])

            @pl.when(s == _NS - 1)
            def _zero_last():
                pltpu.sync_copy(zeros_hbm.at[pl.ds(0, _NRL)],
                                acc.at[pl.ds((_NS - 1) * _NROW, _NRL)])

        plsc.subcore_barrier()

        pltpu.async_copy(v_hbm.at[pl.ds(base, _SCH), pl.ds(col, _HC)],
                         vbufs[0], sems[0])

        def outer(g):
            for par in range(2):
                j = g + par
                nb = 1 - par

                @pl.when(j + 1 < _SK)
                def _start_next():
                    pltpu.async_copy(
                        v_hbm.at[pl.ds(base + (j + 1) * _SCH, _SCH),
                                 pl.ds(col, _HC)],
                        vbufs[nb], sems[nb])

                pltpu.make_async_copy(
                    v_hbm.at[pl.ds(base + j * _SCH, _SCH), pl.ds(col, _HC)],
                    vbufs[par], sems[par]).wait()
                pltpu.sync_copy(vbufs[par], acc.at[idx_v.at[j]], add=True)

        pl.loop(0, _SK, step=2)(outer)
        plsc.subcore_barrier()

        @pl.when(s < _NS - 1)
        def _drain_main():
            pltpu.sync_copy(acc.at[pl.ds(s * _NROW, _NROW)],
                            out.at[pl.ds(s * _NROW, _NROW), pl.ds(col, _HC)])

        @pl.when(s == _NS - 1)
        def _drain_last():
            pltpu.sync_copy(
                acc.at[pl.ds((_NS - 1) * _NROW, _NRL)],
                out.at[pl.ds((_NS - 1) * _NROW, _NRL), pl.ds(col, _HC)])

        plsc.subcore_barrier()


def _sc_scatter(v, dsts, zeros_hbm, init=None):
    out_sds = jax.ShapeDtypeStruct((_N, _H), jnp.float32)
    chained = init is not None
    k = functools.partial(
        pl.kernel,
        out_type=out_sds,
        mesh=_sc_mesh(),
        scratch_types=[
            pltpu.VMEM((_SK, _SCH), jnp.int32),
            pltpu.VMEM((_SCH, _HC), jnp.float32),
            pltpu.VMEM((_SCH, _HC), jnp.float32),
            pltpu.VMEM_SHARED((_NACC, _HC), jnp.float32),
            pltpu.SemaphoreType.DMA,
            pltpu.SemaphoreType.DMA,
        ],
    )(functools.partial(_scatter_body, chained))
    if chained:
        return k(v, dsts, zeros_hbm, init)
    return k(v, dsts, zeros_hbm)


def kernel(x, edge_index, W_in, b_in,
           blk0_W1, blk0_b1, blk0_W2, blk0_b2, blk0_W3, blk0_b3,
           blk1_W1, blk1_b1, blk1_W2, blk1_b2, blk1_W3, blk1_b3,
           W_out, b_out):
    src = edge_index[0]
    dst = edge_index[1]
    pad = _EP - _E
    srcg = jnp.concatenate(
        [src, jnp.zeros((pad,), jnp.int32)]).reshape(2, _EH // _GCH, _GCH)
    dstg = jnp.concatenate(
        [dst, jnp.zeros((pad,), jnp.int32)]).reshape(2, _EH // _GCH, _GCH)
    dsts = jnp.concatenate(
        [dst, jnp.full((pad,), _N, jnp.int32)]).reshape(2, _NS, _SK, _SCH)
    zeros_hbm = jnp.zeros((_NROW, _HC), jnp.float32)

    h0 = _mm_bias([x], W_in, b_in)

    parts = None
    for (W1, b1, W2, b2, W3, b3) in (
            (blk0_W1, blk0_b1, blk0_W2, blk0_b2, blk0_W3, blk0_b3),
            (blk1_W1, blk1_b1, blk1_W2, blk1_b2, blk1_W3, blk1_b3)):
        hin = [h0] if parts is None else list(parts)
        A, B = _mm_ab(hin, W1, b1)
        aj0, bi0 = _sc_gather(A, B, srcg[0], dstg[0])
        v0 = _edge_mlp(aj0, bi0, W2, b2, W3, b3)
        aj1, bi1 = _sc_gather(A, B, srcg[1], dstg[1])
        v1 = _edge_mlp(aj1, bi1, W2, b2, W3, b3)
        p0 = _sc_scatter(v0, dsts[0], zeros_hbm)
        parts = [_sc_scatter(v1, dsts[1], zeros_hbm, init=p0)]

    return _mm_bias(parts, W_out, b_out)


# speedup vs baseline: 4.7172x; 1.0077x over previous
import functools

import jax
import jax.numpy as jnp
from jax import lax
from jax.experimental import pallas as pl
from jax.experimental.pallas import tpu as pltpu
from jax.experimental.pallas import tpu_sc as plsc

_N = 10000
_E = 160000
_H = 512
_NC = 2
_NS = 16
_NW = _NC * _NS
_EP = 163840
_EH = _EP // 2
_GCH = 64
_GPT = _EH // _GCH // _NS
_HH = _H // 2
_HQ = _HH // 2
_SCH = 128
_SK = 40
_EW = _SK * _SCH
_CPC = _H // _NC
_HC = 128
_NPASS = _H // _HC
_NROW = 632
_NRL = _N - 15 * _NROW
_NACC = _N + 16
_TE = 4096
_TM = 2000


def _sc_mesh():
    return plsc.VectorSubcoreMesh(core_axis_name="c", subcore_axis_name="s")


def _mm_body(*refs):
    n_parts = len(refs) - 3
    w, b, o = refs[n_parts], refs[n_parts + 1], refs[n_parts + 2]
    h = refs[0][...]
    for r in refs[1:n_parts]:
        h = h + r[...]
    o[...] = jnp.dot(h, w[...], preferred_element_type=jnp.float32) + b[...]


def _mm_bias(parts, W, bvec):
    M, K = parts[0].shape
    Nc = W.shape[1]
    part_specs = [pl.BlockSpec((_TM, K), lambda i: (i, 0)) for _ in parts]
    return pl.pallas_call(
        _mm_body,
        grid=(M // _TM,),
        in_specs=part_specs + [
            pl.BlockSpec((K, Nc), lambda i: (0, 0)),
            pl.BlockSpec((1, Nc), lambda i: (0, 0)),
        ],
        out_specs=pl.BlockSpec((_TM, Nc), lambda i: (i, 0)),
        out_shape=jax.ShapeDtypeStruct((M, Nc), jnp.float32),
    )(*parts, W, bvec.reshape(1, Nc))


def _pack_bf16(x32):
    half = x32.shape[1] // 2
    u = jax.lax.bitcast_convert_type(x32.astype(jnp.bfloat16),
                                     jnp.uint16).astype(jnp.uint32)
    w = u[:, :half] | (u[:, half:] << 16)
    return jax.lax.bitcast_convert_type(w, jnp.int32)


def _unpack_bf16(p):
    u = jax.lax.bitcast_convert_type(p, jnp.uint32)
    lo = jax.lax.bitcast_convert_type((u & 0xFFFF).astype(jnp.uint16),
                                      jnp.bfloat16)
    hi = jax.lax.bitcast_convert_type((u >> 16).astype(jnp.uint16),
                                      jnp.bfloat16)
    return lo, hi


def _ab_body(*refs):
    n_parts = len(refs) - 6
    wt, wb, bt, bb, a, b = refs[n_parts:]
    h = refs[0][...]
    for r in refs[1:n_parts]:
        h = h + r[...]
    a[...] = _pack_bf16(jnp.dot(h, wt[...],
                                preferred_element_type=jnp.float32) + bt[...])
    b[...] = _pack_bf16(jnp.dot(h, wb[...],
                                preferred_element_type=jnp.float32) + bb[...])


def _mm_ab(parts, Wt, Wb, bt, bb):
    M, K = parts[0].shape
    part_specs = [pl.BlockSpec((_TM, K), lambda i: (i, 0)) for _ in parts]
    out_sds = jax.ShapeDtypeStruct((M, _HH), jnp.int32)
    return pl.pallas_call(
        _ab_body,
        grid=(M // _TM,),
        in_specs=part_specs + [
            pl.BlockSpec((K, _H), lambda i: (0, 0)),
            pl.BlockSpec((K, _H), lambda i: (0, 0)),
            pl.BlockSpec((1, _H), lambda i: (0, 0)),
            pl.BlockSpec((1, _H), lambda i: (0, 0)),
        ],
        out_specs=[pl.BlockSpec((_TM, _HH), lambda i: (i, 0)),
                   pl.BlockSpec((_TM, _HH), lambda i: (i, 0))],
        out_shape=[out_sds, out_sds],
    )(*parts, Wt, Wb, bt.reshape(1, _H), bb.reshape(1, _H))


def _fold_body(win, w1t, w1b, bin_, b1, wt, wb, bt, bb):
    w = win[...]
    bv = bin_[...]
    wt[...] = jnp.dot(w, w1t[...], preferred_element_type=jnp.float32)
    wb[...] = jnp.dot(w, w1b[...], preferred_element_type=jnp.float32)
    bt[...] = jnp.dot(bv, w1t[...], preferred_element_type=jnp.float32) + b1[...]
    bb[...] = jnp.dot(bv, w1b[...], preferred_element_type=jnp.float32)


def _fold_in(W_in, b_in, W1, b1):
    K = W_in.shape[0]
    return pl.pallas_call(
        _fold_body,
        out_shape=[jax.ShapeDtypeStruct((K, _H), jnp.float32),
                   jax.ShapeDtypeStruct((K, _H), jnp.float32),
                   jax.ShapeDtypeStruct((1, _H), jnp.float32),
                   jax.ShapeDtypeStruct((1, _H), jnp.float32)],
    )(W_in, W1[:_H], W1[_H:], b_in.reshape(1, _H), b1.reshape(1, _H))


def _mlp_body(aj, bi, w2, b2, w3, b3, v):
    aj_lo, aj_hi = _unpack_bf16(aj[...])
    bi_lo, bi_hi = _unpack_bf16(bi[...])
    t_lo = jnp.maximum(aj_lo + bi_lo, jnp.bfloat16(0.0))
    t_hi = jnp.maximum(aj_hi + bi_hi, jnp.bfloat16(0.0))
    w2v = w2[...]
    u = (jnp.dot(t_lo, w2v[:_H // 2], preferred_element_type=jnp.float32)
         + jnp.dot(t_hi, w2v[_H // 2:], preferred_element_type=jnp.float32)
         + b2[...])
    u = jnp.maximum(u, 0.0).astype(jnp.bfloat16)
    v[...] = jnp.dot(u, w3[...], preferred_element_type=jnp.float32) + b3[...]


def _edge_mlp(aj, bi, W2, b2, W3, b3):
    return pl.pallas_call(
        _mlp_body,
        grid=(_EH // _TE,),
        in_specs=[
            pl.BlockSpec((_TE, _HH), lambda i: (i, 0)),
            pl.BlockSpec((_TE, _HH), lambda i: (i, 0)),
            pl.BlockSpec((_H, _H), lambda i: (0, 0)),
            pl.BlockSpec((1, _H), lambda i: (0, 0)),
            pl.BlockSpec((_H, _H), lambda i: (0, 0)),
            pl.BlockSpec((1, _H), lambda i: (0, 0)),
        ],
        out_specs=pl.BlockSpec((_TE, _H), lambda i: (i, 0)),
        out_shape=jax.ShapeDtypeStruct((_EH, _H), jnp.float32),
    )(aj, bi, W2.astype(jnp.bfloat16), b2.reshape(1, _H),
      W3.astype(jnp.bfloat16), b3.reshape(1, _H))


def _gather_body(a_hbm, b_hbm, srcg, dstg, aj_hbm, bi_hbm,
                 idx_v, r0, r1, tbl, s0, s1):
    c = lax.axis_index("c")
    s = lax.axis_index("s")
    bufs = (r0, r1)
    sems = (s0, s1)

    def side(table_hbm, idx_src, out):
        pltpu.sync_copy(idx_src.at[pl.ds(s * _GPT, _GPT)], idx_v)
        for p in range(2):
            @pl.when(s < _NS - 1)
            def _load_main():
                pltpu.sync_copy(
                    table_hbm.at[pl.ds(s * _NROW, _NROW), pl.ds(p * _HQ, _HQ)],
                    tbl.at[pl.ds(s * _NROW, _NROW)])

            @pl.when(s == _NS - 1)
            def _load_last():
                pltpu.sync_copy(
                    table_hbm.at[pl.ds((_NS - 1) * _NROW, _NRL),
                                 pl.ds(p * _HQ, _HQ)],
                    tbl.at[pl.ds((_NS - 1) * _NROW, _NRL)])

            plsc.subcore_barrier()
            pltpu.async_copy(tbl.at[idx_v.at[0]], bufs[0], sems[0])

            def outer(g):
                for par in range(2):
                    j = g + par
                    nb = 1 - par

                    @pl.when(j + 1 < _GPT)
                    def _start_next():
                        pltpu.async_copy(tbl.at[idx_v.at[j + 1]],
                                         bufs[nb], sems[nb])

                    pltpu.make_async_copy(tbl.at[idx_v.at[j]],
                                          bufs[par], sems[par]).wait()
                    pltpu.sync_copy(
                        bufs[par],
                        out.at[pl.ds((s * _GPT + j) * _GCH, _GCH),
                               pl.ds(p * _HQ, _HQ)])

            pl.loop(0, _GPT, step=2)(outer)
            plsc.subcore_barrier()

    @pl.when(c == 0)
    def _side_a():
        side(a_hbm, srcg, aj_hbm)

    @pl.when(c == 1)
    def _side_b():
        side(b_hbm, dstg, bi_hbm)


def _sc_gather(A, B, srcg, dstg):
    out_sds = jax.ShapeDtypeStruct((_EH, _HH), jnp.int32)
    k = functools.partial(
        pl.kernel,
        out_type=(out_sds, out_sds),
        mesh=_sc_mesh(),
        scratch_types=[
            pltpu.VMEM((_GPT, _GCH), jnp.int32),
            pltpu.VMEM((_GCH, _HQ), jnp.int32),
            pltpu.VMEM((_GCH, _HQ), jnp.int32),
            pltpu.VMEM_SHARED((_N, _HQ), jnp.int32),
            pltpu.SemaphoreType.DMA,
            pltpu.SemaphoreType.DMA,
        ],
    )(_gather_body)
    return k(A, B, srcg, dstg)


def _scatter_body(chained, *args):
    if chained:
        (v_hbm, dsts, zeros_hbm, init, out,
         idx_v, vb0, vb1, acc, s0, s1) = args
    else:
        (v_hbm, dsts, zeros_hbm, out,
         idx_v, vb0, vb1, acc, s0, s1) = args
    c = lax.axis_index("c")
    s = lax.axis_index("s")
    base = s * _EW
    pltpu.sync_copy(dsts.at[s], idx_v)
    vbufs = (vb0, vb1)
    sems = (s0, s1)

    for p in range(_H // _CPC):
        col = c * _CPC + p * _HC
        if chained:
            @pl.when(s < _NS - 1)
            def _init_main():
                pltpu.sync_copy(
                    init.at[pl.ds(s * _NROW, _NROW), pl.ds(col, _HC)],
                    acc.at[pl.ds(s * _NROW, _NROW)])

            @pl.when(s == _NS - 1)
            def _init_last():
                pltpu.sync_copy(
                    init.at[pl.ds((_NS - 1) * _NROW, _NRL), pl.ds(col, _HC)],
                    acc.at[pl.ds((_NS - 1) * _NROW, _NRL)])
        else:
            @pl.when(s < _NS - 1)
            def _zero_main():
                pltpu.sync_copy(zeros_hbm, acc.at[pl.ds(s * _NROW, _NROW)])

            @pl.when(s == _NS - 1)
            def _zero_last():
                pltpu.sync_copy(zeros_hbm.at[pl.ds(0, _NRL)],
                                acc.at[pl.ds((_NS - 1) * _NROW, _NRL)])

        plsc.subcore_barrier()

        pltpu.async_copy(v_hbm.at[pl.ds(base, _SCH), pl.ds(col, _HC)],
                         vbufs[0], sems[0])

        def outer(g):
            for par in range(2):
                j = g + par
                nb = 1 - par

                @pl.when(j + 1 < _SK)
                def _start_next():
                    pltpu.async_copy(
                        v_hbm.at[pl.ds(base + (j + 1) * _SCH, _SCH),
                                 pl.ds(col, _HC)],
                        vbufs[nb], sems[nb])

                pltpu.make_async_copy(
                    v_hbm.at[pl.ds(base + j * _SCH, _SCH), pl.ds(col, _HC)],
                    vbufs[par], sems[par]).wait()
                pltpu.sync_copy(vbufs[par], acc.at[idx_v.at[j]], add=True)

        pl.loop(0, _SK, step=2)(outer)
        plsc.subcore_barrier()

        @pl.when(s < _NS - 1)
        def _drain_main():
            pltpu.sync_copy(acc.at[pl.ds(s * _NROW, _NROW)],
                            out.at[pl.ds(s * _NROW, _NROW), pl.ds(col, _HC)])

        @pl.when(s == _NS - 1)
        def _drain_last():
            pltpu.sync_copy(
                acc.at[pl.ds((_NS - 1) * _NROW, _NRL)],
                out.at[pl.ds((_NS - 1) * _NROW, _NRL), pl.ds(col, _HC)])

        plsc.subcore_barrier()


def _sc_scatter(v, dsts, zeros_hbm, init=None):
    out_sds = jax.ShapeDtypeStruct((_N, _H), jnp.float32)
    chained = init is not None
    k = functools.partial(
        pl.kernel,
        out_type=out_sds,
        mesh=_sc_mesh(),
        scratch_types=[
            pltpu.VMEM((_SK, _SCH), jnp.int32),
            pltpu.VMEM((_SCH, _HC), jnp.float32),
            pltpu.VMEM((_SCH, _HC), jnp.float32),
            pltpu.VMEM_SHARED((_NACC, _HC), jnp.float32),
            pltpu.SemaphoreType.DMA,
            pltpu.SemaphoreType.DMA,
        ],
    )(functools.partial(_scatter_body, chained))
    if chained:
        return k(v, dsts, zeros_hbm, init)
    return k(v, dsts, zeros_hbm)


def kernel(x, edge_index, W_in, b_in,
           blk0_W1, blk0_b1, blk0_W2, blk0_b2, blk0_W3, blk0_b3,
           blk1_W1, blk1_b1, blk1_W2, blk1_b2, blk1_W3, blk1_b3,
           W_out, b_out):
    src = edge_index[0]
    dst = edge_index[1]
    pad = _EP - _E
    srcg = jnp.concatenate(
        [src, jnp.zeros((pad,), jnp.int32)]).reshape(2, _EH // _GCH, _GCH)
    dstg = jnp.concatenate(
        [dst, jnp.zeros((pad,), jnp.int32)]).reshape(2, _EH // _GCH, _GCH)
    dsts = jnp.concatenate(
        [dst, jnp.full((pad,), _N, jnp.int32)]).reshape(2, _NS, _SK, _SCH)
    zeros_hbm = jnp.zeros((_NROW, _HC), jnp.float32)

    fw_t, fw_b, fb_t, fb_b = _fold_in(W_in, b_in, blk0_W1, blk0_b1)
    zcol = jnp.zeros((1, _H), jnp.float32)

    parts = None
    for (W1, b1, W2, b2, W3, b3) in (
            (blk0_W1, blk0_b1, blk0_W2, blk0_b2, blk0_W3, blk0_b3),
            (blk1_W1, blk1_b1, blk1_W2, blk1_b2, blk1_W3, blk1_b3)):
        if parts is None:
            hin = [x]
            A, B = _mm_ab(hin, fw_t, fw_b, fb_t, fb_b)
        else:
            hin = list(parts)
            A, B = _mm_ab(hin, W1[:_H], W1[_H:], b1, jnp.zeros((_H,)))
        aj0, bi0 = _sc_gather(A, B, srcg[0], dstg[0])
        v0 = _edge_mlp(aj0, bi0, W2, b2, W3, b3)
        aj1, bi1 = _sc_gather(A, B, srcg[1], dstg[1])
        v1 = _edge_mlp(aj1, bi1, W2, b2, W3, b3)
        p0 = _sc_scatter(v0, dsts[0], zeros_hbm)
        parts = [_sc_scatter(v1, dsts[1], zeros_hbm, init=p0)]

    return _mm_bias(parts, W_out, b_out)
